# Initial kernel scaffold; baseline (speedup 1.0000x reference)
import jax
import jax.numpy as jnp
from jax.experimental import pallas as pl


def kernel(x, W_f4_self, b_f4_self, W_f4_e0, b_f4_e0, W_f4_e1, b_f4_e1, W_gr_self, b_gr_self, W_gr_e0, b_gr_e0, W_gr_e1, b_gr_e1, W_st_self, b_st_self, W_st_e0, b_st_e0, W_st_e1, b_st_e1):
    raise NotImplementedError("write your pallas kernel here")


# speedup vs baseline: 4.6800x; 4.6800x over previous
import functools

import jax
import jax.numpy as jnp
from jax import lax
from jax.experimental import pallas as pl
from jax.experimental.pallas import tpu as pltpu
from jax.experimental.pallas import tpu_sc as plsc

NW = 32
GS = 16
CG = 25
CS = CG * GS


def _sc_body(ns, x_hbm, w_hbm, y4_hbm, ygr_hbm, yst_hbm,
             xv, wv, y4v, ygrv, ystv):
    cid = lax.axis_index("c")
    sid = lax.axis_index("s")
    wid = sid * 2 + cid
    nq_total = ns // CS
    nq = (nq_total - wid + NW - 1) // NW

    pltpu.sync_copy(w_hbm, wv)
    iota = lax.iota(jnp.int32, GS)

    def wrow(r):
        return wv[pl.ds(r * GS, GS)]

    def chunk_body(i, carry):
        q = wid + i * NW
        pltpu.sync_copy(x_hbm.at[pl.ds(q * CS * 24, CS * 24)], xv)

        def group_body(g, c2):
            sidx = g * GS + iota
            i24 = sidx * 24
            i6 = sidx * 6
            X = [[plsc.load_gather(xv, [i24 + (c * 6 + n)])
                  for c in range(4)] for n in range(6)]
            S = [[X[3 * h][c] + X[3 * h + 1][c] + X[3 * h + 2][c]
                  for c in range(4)] for h in range(2)]
            U = []
            for h in range(2):
                row = []
                for o in range(6):
                    acc = wrow(72 + o)
                    for c in range(4):
                        acc = acc + wrow(48 + o * 4 + c) * S[h][c]
                    row.append(acc)
                U.append(row)
            for n in range(6):
                h = n // 3
                p = (n + 3) % 6
                for o in range(6):
                    acc = U[h][o]
                    for c in range(4):
                        acc = acc + wrow(o * 4 + c) * X[n][c]
                    for c in range(4):
                        acc = acc + wrow(24 + o * 4 + c) * X[p][c]
                    if o < 4:
                        plsc.store_scatter(y4v, [i24 + (n * 4 + o)], acc)
                    elif o == 4:
                        plsc.store_scatter(ygrv, [i6 + n], acc)
                    else:
                        plsc.store_scatter(ystv, [i6 + n], acc)
            return c2

        lax.fori_loop(0, CG, group_body, 0)
        pltpu.sync_copy(y4v, y4_hbm.at[pl.ds(q * CS * 24, CS * 24)])
        pltpu.sync_copy(ygrv, ygr_hbm.at[pl.ds(q * CS * 6, CS * 6)])
        pltpu.sync_copy(ystv, yst_hbm.at[pl.ds(q * CS * 6, CS * 6)])
        return carry

    lax.fori_loop(0, nq, chunk_body, 0)


def kernel(x,
           W_f4_self, b_f4_self, W_f4_e0, b_f4_e0, W_f4_e1, b_f4_e1,
           W_gr_self, b_gr_self, W_gr_e0, b_gr_e0, W_gr_e1, b_gr_e1,
           W_st_self, b_st_self, W_st_e0, b_st_e0, W_st_e1, b_st_e1):
    ns = x.shape[0]
    f32 = jnp.float32

    Wself = jnp.concatenate([W_f4_self, W_gr_self, W_st_self], axis=0)
    We0 = jnp.concatenate([W_f4_e0, W_gr_e0, W_st_e0], axis=0)
    We1 = jnp.concatenate([W_f4_e1, W_gr_e1, W_st_e1], axis=0)
    bs = jnp.concatenate([b_f4_self, b_gr_self, b_st_self])
    b0 = jnp.concatenate([b_f4_e0, b_gr_e0, b_st_e0])
    b1 = jnp.concatenate([b_f4_e1, b_gr_e1, b_st_e1])
    A = Wself - We0
    B = We0
    C = We1
    d = bs + 2.0 * b0 + b1
    rows = jnp.concatenate(
        [A.reshape(24), C.reshape(24), B.reshape(24), d]).astype(f32)
    wbig = jnp.tile(rows[:, None], (1, GS)).reshape(78 * GS)

    x2 = x.astype(f32).reshape(ns * 24)

    mesh = plsc.VectorSubcoreMesh(core_axis_name="c", subcore_axis_name="s",
                                  num_cores=2, num_subcores=16)
    run = pl.kernel(
        functools.partial(_sc_body, ns),
        out_type=(jax.ShapeDtypeStruct((ns * 24,), f32),
                  jax.ShapeDtypeStruct((ns * 6,), f32),
                  jax.ShapeDtypeStruct((ns * 6,), f32)),
        mesh=mesh,
        compiler_params=pltpu.CompilerParams(needs_layout_passes=False),
        scratch_types=(pltpu.VMEM((CS * 24,), f32),
                       pltpu.VMEM((78 * GS,), f32),
                       pltpu.VMEM((CS * 24,), f32),
                       pltpu.VMEM((CS * 6,), f32),
                       pltpu.VMEM((CS * 6,), f32)),
    )
    y4, ygr, yst = run(x2, wbig)
    return (y4.reshape(ns * 6, 4),
            ygr.reshape(ns * 6, 1),
            yst.reshape(ns * 6, 1))


# speedup vs baseline: 144.8626x; 30.9536x over previous
import functools

import jax
import jax.numpy as jnp
from jax import lax
from jax.experimental import pallas as pl
from jax.experimental.pallas import tpu as pltpu
from jax.experimental.pallas import tpu_sc as plsc

NW = 32
GS = 16
CG = 50
CS = CG * GS


def _feat(n, c):
    return c * 6 + (n % 3) * 2 + (n // 3)


def _sc_body(ns, x_hbm, w_hbm, y4_hbm, ygr_hbm, yst_hbm,
             xv0, xv1, wv, y4v0, y4v1, ygrv0, ygrv1, ystv0, ystv1,
             isem0, isem1, osem0, osem1):
    cid = lax.axis_index("c")
    sid = lax.axis_index("s")
    wid = sid * 2 + cid
    nq_total = ns // CS
    nq = (nq_total - wid + NW - 1) // NW

    pltpu.sync_copy(w_hbm, wv)
    iota = lax.iota(jnp.int32, GS)

    bufs = ((xv0, y4v0, ygrv0, ystv0, isem0, osem0),
            (xv1, y4v1, ygrv1, ystv1, isem1, osem1))

    def wrow(r):
        return wv[pl.ds(r * GS, GS)]

    def _in_copies(q, xb, sem):
        base = q * CS
        return [pltpu.make_async_copy(x_hbm.at[pl.ds(f * ns + base, CS)],
                                      xb.at[pl.ds(f * CS, CS)], sem)
                for f in range(24)]

    def _out_copies(q, y4b, grb, stb, sem):
        cps = [pltpu.make_async_copy(
                   y4b.at[pl.ds(c * CS * 6, CS * 6)],
                   y4_hbm.at[pl.ds(c * (ns * 6) + q * CS * 6, CS * 6)], sem)
               for c in range(4)]
        cps.append(pltpu.make_async_copy(
            grb, ygr_hbm.at[pl.ds(q * CS * 6, CS * 6)], sem))
        cps.append(pltpu.make_async_copy(
            stb, yst_hbm.at[pl.ds(q * CS * 6, CS * 6)], sem))
        return cps

    def compute_chunk(xb, y4b, grb, stb):

        @plsc.parallel_loop(0, CG, 1, unroll=4)
        def group_body(g):
            sidx = g * GS + iota
            i6 = sidx * 6
            X = [[xb[pl.ds(_feat(n, c) * CS + g * GS, GS)]
                  for c in range(4)] for n in range(6)]
            S = [[X[3 * h][c] + X[3 * h + 1][c] + X[3 * h + 2][c]
                  for c in range(4)] for h in range(2)]
            U = []
            for h in range(2):
                row = []
                for o in range(6):
                    acc = wrow(72 + o)
                    for c in range(4):
                        acc = acc + wrow(48 + o * 4 + c) * S[h][c]
                    row.append(acc)
                U.append(row)

            def put(n, o, acc):
                if o < 4:
                    plsc.store_scatter(y4b, [i6 + (o * (CS * 6) + n)], acc)
                elif o == 4:
                    plsc.store_scatter(grb, [i6 + n], acc)
                else:
                    plsc.store_scatter(stb, [i6 + n], acc)

            for ho in (0, 3):
                Ao = {(o, c): wrow(o * 4 + c)
                      for o in (ho, ho + 1, ho + 2) for c in range(4)}
                Co = {(o, c): wrow(24 + o * 4 + c)
                      for o in (ho, ho + 1, ho + 2) for c in range(4)}
                for j in range(3):
                    Xa = [xb[pl.ds(_feat(j, c) * CS + g * GS, GS)]
                          for c in range(4)]
                    Xp = [xb[pl.ds(_feat(j + 3, c) * CS + g * GS, GS)]
                          for c in range(4)]
                    for o in (ho, ho + 1, ho + 2):
                        a = U[0][o]
                        b = U[1][o]
                        for c in range(4):
                            a = a + Ao[o, c] * Xa[c]
                            b = b + Ao[o, c] * Xp[c]
                        for c in range(4):
                            a = a + Co[o, c] * Xp[c]
                            b = b + Co[o, c] * Xa[c]
                        put(j, o, a)
                        put(j + 3, o, b)

    for k in (0, 1):
        xb, _, _, _, isem, _ = bufs[k]

        @pl.when(k < nq)
        def _(k=k, xb=xb, isem=isem):
            for cp in _in_copies(wid + k * NW, xb, isem):
                cp.start()

    def pair_body(ip, carry):
        for k in (0, 1):
            xb, y4b, grb, stb, isem, osem = bufs[k]
            i = ip * 2 + k

            @pl.when(i < nq)
            def _(i=i, xb=xb, y4b=y4b, grb=grb, stb=stb,
                  isem=isem, osem=osem):
                q = wid + i * NW
                for cp in _in_copies(q, xb, isem):
                    cp.wait()

                @pl.when(i >= 2)
                def _():
                    for cp in _out_copies(wid + (i - 2) * NW,
                                          y4b, grb, stb, osem):
                        cp.wait()

                compute_chunk(xb, y4b, grb, stb)
                for cp in _out_copies(q, y4b, grb, stb, osem):
                    cp.start()

                @pl.when(i + 2 < nq)
                def _():
                    for cp in _in_copies(wid + (i + 2) * NW, xb, isem):
                        cp.start()
        return carry

    lax.fori_loop(0, (nq + 1) // 2, pair_body, 0)

    for k in (0, 1):
        _, y4b, grb, stb, _, osem = bufs[k]
        ik = ((nq - 1 - k) // 2) * 2 + k

        @pl.when(ik >= 0)
        def _(ik=ik, y4b=y4b, grb=grb, stb=stb, osem=osem):
            for cp in _out_copies(wid + ik * NW, y4b, grb, stb, osem):
                cp.wait()


def kernel(x,
           W_f4_self, b_f4_self, W_f4_e0, b_f4_e0, W_f4_e1, b_f4_e1,
           W_gr_self, b_gr_self, W_gr_e0, b_gr_e0, W_gr_e1, b_gr_e1,
           W_st_self, b_st_self, W_st_e0, b_st_e0, W_st_e1, b_st_e1):
    ns = x.shape[0]
    f32 = jnp.float32

    Wself = jnp.concatenate([W_f4_self, W_gr_self, W_st_self], axis=0)
    We0 = jnp.concatenate([W_f4_e0, W_gr_e0, W_st_e0], axis=0)
    We1 = jnp.concatenate([W_f4_e1, W_gr_e1, W_st_e1], axis=0)
    bs = jnp.concatenate([b_f4_self, b_gr_self, b_st_self])
    b0 = jnp.concatenate([b_f4_e0, b_gr_e0, b_st_e0])
    b1 = jnp.concatenate([b_f4_e1, b_gr_e1, b_st_e1])
    A = Wself - We0
    B = We0
    C = We1
    d = bs + 2.0 * b0 + b1
    rows = jnp.concatenate(
        [A.reshape(24), C.reshape(24), B.reshape(24), d]).astype(f32)
    wbig = jnp.tile(rows[:, None], (1, GS)).reshape(78 * GS)

    x2 = jnp.transpose(x.astype(f32), (1, 3, 2, 0)).reshape(24 * ns)

    mesh = plsc.VectorSubcoreMesh(core_axis_name="c", subcore_axis_name="s",
                                  num_cores=2, num_subcores=16)
    run = pl.kernel(
        functools.partial(_sc_body, ns),
        out_type=(jax.ShapeDtypeStruct((ns * 24,), f32),
                  jax.ShapeDtypeStruct((ns * 6,), f32),
                  jax.ShapeDtypeStruct((ns * 6,), f32)),
        mesh=mesh,
        compiler_params=pltpu.CompilerParams(needs_layout_passes=False),
        scratch_types=(pltpu.VMEM((CS * 24,), f32),
                       pltpu.VMEM((CS * 24,), f32),
                       pltpu.VMEM((78 * GS,), f32),
                       pltpu.VMEM((CS * 24,), f32),
                       pltpu.VMEM((CS * 24,), f32),
                       pltpu.VMEM((CS * 6,), f32),
                       pltpu.VMEM((CS * 6,), f32),
                       pltpu.VMEM((CS * 6,), f32),
                       pltpu.VMEM((CS * 6,), f32),
                       pltpu.SemaphoreType.DMA,
                       pltpu.SemaphoreType.DMA,
                       pltpu.SemaphoreType.DMA,
                       pltpu.SemaphoreType.DMA),
    )
    y4, ygr, yst = run(x2, wbig)
    return (y4.reshape(4, ns * 6).T,
            ygr.reshape(ns * 6, 1),
            yst.reshape(ns * 6, 1))


# speedup vs baseline: 158.0572x; 1.0911x over previous
import functools

import jax
import jax.numpy as jnp
from jax import lax
from jax.experimental import pallas as pl
from jax.experimental.pallas import tpu as pltpu
from jax.experimental.pallas import tpu_sc as plsc

NW = 32
GS = 16
CG = 50
CS = CG * GS


def _feat(n, c):
    return c * 6 + (n % 3) * 2 + (n // 3)


def _sc_body(ns, x_hbm, w_hbm, y4_hbm, ygr_hbm, yst_hbm,
             xv0, xv1, wv, y4v0, y4v1, ygrv0, ygrv1, ystv0, ystv1,
             isem0, isem1, osem0, osem1):
    cid = lax.axis_index("c")
    sid = lax.axis_index("s")
    wid = sid * 2 + cid
    nq_total = ns // CS
    nq = (nq_total - wid + NW - 1) // NW

    pltpu.sync_copy(w_hbm, wv)
    iota = lax.iota(jnp.int32, GS)

    bufs = ((xv0, y4v0, ygrv0, ystv0, isem0, osem0),
            (xv1, y4v1, ygrv1, ystv1, isem1, osem1))

    def wrow(r):
        return wv[pl.ds(r * GS, GS)]

    def _in_copies(q, xb, sem):
        base = q * CS
        return [pltpu.make_async_copy(x_hbm.at[pl.ds(f * ns + base, CS)],
                                      xb.at[pl.ds(f * CS, CS)], sem)
                for f in range(24)]

    def _out_copies(q, y4b, grb, stb, sem):
        cps = [pltpu.make_async_copy(
                   y4b.at[pl.ds(c * CS * 6, CS * 6)],
                   y4_hbm.at[pl.ds(c * (ns * 6) + q * CS * 6, CS * 6)], sem)
               for c in range(4)]
        cps.append(pltpu.make_async_copy(
            grb, ygr_hbm.at[pl.ds(q * CS * 6, CS * 6)], sem))
        cps.append(pltpu.make_async_copy(
            stb, yst_hbm.at[pl.ds(q * CS * 6, CS * 6)], sem))
        return cps

    def compute_chunk(xb, y4b, grb, stb):

        @plsc.parallel_loop(0, CG, 1, unroll=2)
        def group_body(g):
            sidx = g * GS + iota
            i6 = sidx * 6
            X = [[xb[pl.ds(_feat(n, c) * CS + g * GS, GS)]
                  for c in range(4)] for n in range(6)]
            S = [[X[3 * h][c] + X[3 * h + 1][c] + X[3 * h + 2][c]
                  for c in range(4)] for h in range(2)]
            U = []
            for h in range(2):
                row = []
                for o in range(6):
                    acc = wrow(72 + o)
                    for c in range(4):
                        acc = acc + wrow(48 + o * 4 + c) * S[h][c]
                    row.append(acc)
                U.append(row)

            def put(n, o, acc):
                if o < 4:
                    plsc.store_scatter(y4b, [i6 + (o * (CS * 6) + n)], acc)
                elif o == 4:
                    plsc.store_scatter(grb, [i6 + n], acc)
                else:
                    plsc.store_scatter(stb, [i6 + n], acc)

            for ho in (0, 3):
                Ao = {(o, c): wrow(o * 4 + c)
                      for o in (ho, ho + 1, ho + 2) for c in range(4)}
                Co = {(o, c): wrow(24 + o * 4 + c)
                      for o in (ho, ho + 1, ho + 2) for c in range(4)}
                for j in range(3):
                    Xa = [xb[pl.ds(_feat(j, c) * CS + g * GS, GS)]
                          for c in range(4)]
                    Xp = [xb[pl.ds(_feat(j + 3, c) * CS + g * GS, GS)]
                          for c in range(4)]
                    for o in (ho, ho + 1, ho + 2):
                        a = U[0][o]
                        b = U[1][o]
                        for c in range(4):
                            a = a + Ao[o, c] * Xa[c]
                            b = b + Ao[o, c] * Xp[c]
                        for c in range(4):
                            a = a + Co[o, c] * Xp[c]
                            b = b + Co[o, c] * Xa[c]
                        put(j, o, a)
                        put(j + 3, o, b)

    for k in (0, 1):
        xb, _, _, _, isem, _ = bufs[k]

        @pl.when(k < nq)
        def _(k=k, xb=xb, isem=isem):
            for cp in _in_copies(wid + k * NW, xb, isem):
                cp.start()

    def pair_body(ip, carry):
        for k in (0, 1):
            xb, y4b, grb, stb, isem, osem = bufs[k]
            i = ip * 2 + k

            @pl.when(i < nq)
            def _(i=i, xb=xb, y4b=y4b, grb=grb, stb=stb,
                  isem=isem, osem=osem):
                q = wid + i * NW
                for cp in _in_copies(q, xb, isem):
                    cp.wait()

                @pl.when(i >= 2)
                def _():
                    for cp in _out_copies(wid + (i - 2) * NW,
                                          y4b, grb, stb, osem):
                        cp.wait()

                compute_chunk(xb, y4b, grb, stb)
                for cp in _out_copies(q, y4b, grb, stb, osem):
                    cp.start()

                @pl.when(i + 2 < nq)
                def _():
                    for cp in _in_copies(wid + (i + 2) * NW, xb, isem):
                        cp.start()
        return carry

    lax.fori_loop(0, (nq + 1) // 2, pair_body, 0)

    for k in (0, 1):
        _, y4b, grb, stb, _, osem = bufs[k]
        ik = ((nq - 1 - k) // 2) * 2 + k

        @pl.when(ik >= 0)
        def _(ik=ik, y4b=y4b, grb=grb, stb=stb, osem=osem):
            for cp in _out_copies(wid + ik * NW, y4b, grb, stb, osem):
                cp.wait()


def kernel(x,
           W_f4_self, b_f4_self, W_f4_e0, b_f4_e0, W_f4_e1, b_f4_e1,
           W_gr_self, b_gr_self, W_gr_e0, b_gr_e0, W_gr_e1, b_gr_e1,
           W_st_self, b_st_self, W_st_e0, b_st_e0, W_st_e1, b_st_e1):
    ns = x.shape[0]
    f32 = jnp.float32

    Wself = jnp.concatenate([W_f4_self, W_gr_self, W_st_self], axis=0)
    We0 = jnp.concatenate([W_f4_e0, W_gr_e0, W_st_e0], axis=0)
    We1 = jnp.concatenate([W_f4_e1, W_gr_e1, W_st_e1], axis=0)
    bs = jnp.concatenate([b_f4_self, b_gr_self, b_st_self])
    b0 = jnp.concatenate([b_f4_e0, b_gr_e0, b_st_e0])
    b1 = jnp.concatenate([b_f4_e1, b_gr_e1, b_st_e1])
    A = Wself - We0
    B = We0
    C = We1
    d = bs + 2.0 * b0 + b1
    rows = jnp.concatenate(
        [A.reshape(24), C.reshape(24), B.reshape(24), d]).astype(f32)
    wbig = jnp.tile(rows[:, None], (1, GS)).reshape(78 * GS)

    x2 = jnp.transpose(x.astype(f32), (1, 3, 2, 0)).reshape(24 * ns)

    mesh = plsc.VectorSubcoreMesh(core_axis_name="c", subcore_axis_name="s",
                                  num_cores=2, num_subcores=16)
    run = pl.kernel(
        functools.partial(_sc_body, ns),
        out_type=(jax.ShapeDtypeStruct((ns * 24,), f32),
                  jax.ShapeDtypeStruct((ns * 6,), f32),
                  jax.ShapeDtypeStruct((ns * 6,), f32)),
        mesh=mesh,
        compiler_params=pltpu.CompilerParams(needs_layout_passes=False),
        scratch_types=(pltpu.VMEM((CS * 24,), f32),
                       pltpu.VMEM((CS * 24,), f32),
                       pltpu.VMEM((78 * GS,), f32),
                       pltpu.VMEM((CS * 24,), f32),
                       pltpu.VMEM((CS * 24,), f32),
                       pltpu.VMEM((CS * 6,), f32),
                       pltpu.VMEM((CS * 6,), f32),
                       pltpu.VMEM((CS * 6,), f32),
                       pltpu.VMEM((CS * 6,), f32),
                       pltpu.SemaphoreType.DMA,
                       pltpu.SemaphoreType.DMA,
                       pltpu.SemaphoreType.DMA,
                       pltpu.SemaphoreType.DMA),
    )
    y4, ygr, yst = run(x2, wbig)
    return (y4.reshape(4, ns * 6).T,
            ygr.reshape(ns * 6, 1),
            yst.reshape(ns * 6, 1))


# speedup vs baseline: 160.9333x; 1.0182x over previous
import functools

import jax
import jax.numpy as jnp
from jax import lax
from jax.experimental import pallas as pl
from jax.experimental.pallas import tpu as pltpu
from jax.experimental.pallas import tpu_sc as plsc

NW = 32
GS = 16
CG = 50
CS = CG * GS


def _feat(n, c):
    return c * 6 + (n % 3) * 2 + (n // 3)


def _sc_body(ns, x_hbm, w_hbm, y4_hbm, ygr_hbm, yst_hbm,
             xv0, xv1, wv, y4v0, y4v1, ygrv0, ygrv1, ystv0, ystv1,
             isem0, isem1, osem0, osem1):
    cid = lax.axis_index("c")
    sid = lax.axis_index("s")
    wid = sid * 2 + cid
    nq_total = ns // CS
    nq = (nq_total - wid + NW - 1) // NW

    pltpu.sync_copy(w_hbm, wv)
    iota = lax.iota(jnp.int32, GS)

    bufs = ((xv0, y4v0, ygrv0, ystv0, isem0, osem0),
            (xv1, y4v1, ygrv1, ystv1, isem1, osem1))

    def wrow(r):
        return wv[pl.ds(r * GS, GS)]

    def _in_copies(q, xb, sem):
        base = q * CS
        return [pltpu.make_async_copy(x_hbm.at[pl.ds(f * ns + base, CS)],
                                      xb.at[pl.ds(f * CS, CS)], sem)
                for f in range(24)]

    def _out_copies(q, y4b, grb, stb, sem):
        cps = [pltpu.make_async_copy(
                   y4b.at[pl.ds(c * CS * 6, CS * 6)],
                   y4_hbm.at[pl.ds(c * (ns * 6) + q * CS * 6, CS * 6)], sem)
               for c in range(4)]
        cps.append(pltpu.make_async_copy(
            grb, ygr_hbm.at[pl.ds(q * CS * 6, CS * 6)], sem))
        cps.append(pltpu.make_async_copy(
            stb, yst_hbm.at[pl.ds(q * CS * 6, CS * 6)], sem))
        return cps

    def compute_chunk(xb, y4b, grb, stb):

        @plsc.parallel_loop(0, CG, 1, unroll=3)
        def group_body(g):
            sidx = g * GS + iota
            i6 = sidx * 6
            X = [[xb[pl.ds(_feat(n, c) * CS + g * GS, GS)]
                  for c in range(4)] for n in range(6)]
            S = [[X[3 * h][c] + X[3 * h + 1][c] + X[3 * h + 2][c]
                  for c in range(4)] for h in range(2)]
            U = []
            for h in range(2):
                row = []
                for o in range(6):
                    acc = wrow(72 + o)
                    for c in range(4):
                        acc = acc + wrow(48 + o * 4 + c) * S[h][c]
                    row.append(acc)
                U.append(row)

            def put(n, o, acc):
                if o < 4:
                    plsc.store_scatter(y4b, [i6 + (o * (CS * 6) + n)], acc)
                elif o == 4:
                    plsc.store_scatter(grb, [i6 + n], acc)
                else:
                    plsc.store_scatter(stb, [i6 + n], acc)

            for ho in (0, 3):
                Ao = {(o, c): wrow(o * 4 + c)
                      for o in (ho, ho + 1, ho + 2) for c in range(4)}
                Co = {(o, c): wrow(24 + o * 4 + c)
                      for o in (ho, ho + 1, ho + 2) for c in range(4)}
                for j in range(3):
                    Xa = [xb[pl.ds(_feat(j, c) * CS + g * GS, GS)]
                          for c in range(4)]
                    Xp = [xb[pl.ds(_feat(j + 3, c) * CS + g * GS, GS)]
                          for c in range(4)]
                    for o in (ho, ho + 1, ho + 2):
                        a = U[0][o]
                        b = U[1][o]
                        for c in range(4):
                            a = a + Ao[o, c] * Xa[c]
                            b = b + Ao[o, c] * Xp[c]
                        for c in range(4):
                            a = a + Co[o, c] * Xp[c]
                            b = b + Co[o, c] * Xa[c]
                        put(j, o, a)
                        put(j + 3, o, b)

    for k in (0, 1):
        xb, _, _, _, isem, _ = bufs[k]

        @pl.when(k < nq)
        def _(k=k, xb=xb, isem=isem):
            for cp in _in_copies(wid + k * NW, xb, isem):
                cp.start()

    def pair_body(ip, carry):
        for k in (0, 1):
            xb, y4b, grb, stb, isem, osem = bufs[k]
            i = ip * 2 + k

            @pl.when(i < nq)
            def _(i=i, xb=xb, y4b=y4b, grb=grb, stb=stb,
                  isem=isem, osem=osem):
                q = wid + i * NW
                for cp in _in_copies(q, xb, isem):
                    cp.wait()

                @pl.when(i >= 2)
                def _():
                    for cp in _out_copies(wid + (i - 2) * NW,
                                          y4b, grb, stb, osem):
                        cp.wait()

                compute_chunk(xb, y4b, grb, stb)
                for cp in _out_copies(q, y4b, grb, stb, osem):
                    cp.start()

                @pl.when(i + 2 < nq)
                def _():
                    for cp in _in_copies(wid + (i + 2) * NW, xb, isem):
                        cp.start()
        return carry

    lax.fori_loop(0, (nq + 1) // 2, pair_body, 0)

    for k in (0, 1):
        _, y4b, grb, stb, _, osem = bufs[k]
        ik = ((nq - 1 - k) // 2) * 2 + k

        @pl.when(ik >= 0)
        def _(ik=ik, y4b=y4b, grb=grb, stb=stb, osem=osem):
            for cp in _out_copies(wid + ik * NW, y4b, grb, stb, osem):
                cp.wait()


def kernel(x,
           W_f4_self, b_f4_self, W_f4_e0, b_f4_e0, W_f4_e1, b_f4_e1,
           W_gr_self, b_gr_self, W_gr_e0, b_gr_e0, W_gr_e1, b_gr_e1,
           W_st_self, b_st_self, W_st_e0, b_st_e0, W_st_e1, b_st_e1):
    ns = x.shape[0]
    f32 = jnp.float32

    Wself = jnp.concatenate([W_f4_self, W_gr_self, W_st_self], axis=0)
    We0 = jnp.concatenate([W_f4_e0, W_gr_e0, W_st_e0], axis=0)
    We1 = jnp.concatenate([W_f4_e1, W_gr_e1, W_st_e1], axis=0)
    bs = jnp.concatenate([b_f4_self, b_gr_self, b_st_self])
    b0 = jnp.concatenate([b_f4_e0, b_gr_e0, b_st_e0])
    b1 = jnp.concatenate([b_f4_e1, b_gr_e1, b_st_e1])
    A = Wself - We0
    B = We0
    C = We1
    d = bs + 2.0 * b0 + b1
    rows = jnp.concatenate(
        [A.reshape(24), C.reshape(24), B.reshape(24), d]).astype(f32)
    wbig = jnp.tile(rows[:, None], (1, GS)).reshape(78 * GS)

    x2 = jnp.transpose(x.astype(f32), (1, 3, 2, 0)).reshape(24 * ns)

    mesh = plsc.VectorSubcoreMesh(core_axis_name="c", subcore_axis_name="s",
                                  num_cores=2, num_subcores=16)
    run = pl.kernel(
        functools.partial(_sc_body, ns),
        out_type=(jax.ShapeDtypeStruct((ns * 24,), f32),
                  jax.ShapeDtypeStruct((ns * 6,), f32),
                  jax.ShapeDtypeStruct((ns * 6,), f32)),
        mesh=mesh,
        compiler_params=pltpu.CompilerParams(needs_layout_passes=False),
        scratch_types=(pltpu.VMEM((CS * 24,), f32),
                       pltpu.VMEM((CS * 24,), f32),
                       pltpu.VMEM((78 * GS,), f32),
                       pltpu.VMEM((CS * 24,), f32),
                       pltpu.VMEM((CS * 24,), f32),
                       pltpu.VMEM((CS * 6,), f32),
                       pltpu.VMEM((CS * 6,), f32),
                       pltpu.VMEM((CS * 6,), f32),
                       pltpu.VMEM((CS * 6,), f32),
                       pltpu.SemaphoreType.DMA,
                       pltpu.SemaphoreType.DMA,
                       pltpu.SemaphoreType.DMA,
                       pltpu.SemaphoreType.DMA),
    )
    y4, ygr, yst = run(x2, wbig)
    return (y4.reshape(4, ns * 6).T,
            ygr.reshape(ns * 6, 1),
            yst.reshape(ns * 6, 1))
